# baseline (speedup 1.0000x reference)
import functools

import jax
import jax.numpy as jnp
from jax import lax
from jax.experimental import pallas as pl
from jax.experimental.pallas import tpu as pltpu
from jax.experimental.pallas import tpu_sc as plsc

N = 10000
NP = 10240
E = 320000
DIN = 128
F1 = 512
F2 = 256
F3 = 64
G = 64
NC = 10

NCORES = 2
NSUB = 16
NW = NCORES * NSUB
K = 128
CH = 2560
EP = CH * K
ROWS_PT = NP // NSUB


def _sc_mesh():
    return plsc.VectorSubcoreMesh(core_axis_name="c", subcore_axis_name="s")


DW = 128


DEG_CPT = CH // NW


@functools.partial(
    pl.kernel,
    out_type=[
        jax.ShapeDtypeStruct((NP, DW), jnp.float32),
        jax.ShapeDtypeStruct((NP, DW), jnp.float32),
    ],
    mesh=_sc_mesh(),
    scratch_types=[
        pltpu.VMEM((DEG_CPT, K), jnp.int32),
        pltpu.VMEM((K, DW), jnp.float32),
        pltpu.VMEM_SHARED((NP, DW), jnp.float32),
        pltpu.SemaphoreType.DMA,
    ],
)
def _deg_kernel(dst2_hbm, out0_hbm, out1_hbm, dsts_v, ones_v, acc_sh, ssem):
    cid = lax.axis_index("c")
    sid = lax.axis_index("s")
    wid = cid * NSUB + sid

    def zero_body(i, carry):
        for jj in range(DW // 16):
            ones_v[i, pl.ds(jj * 16, 16)] = jnp.zeros((16,), jnp.float32)
        return carry

    lax.fori_loop(0, K, zero_body, 0)
    r0 = sid * ROWS_PT
    for t in range(ROWS_PT // K):
        pltpu.sync_copy(ones_v, acc_sh.at[pl.ds(r0 + t * K, K)])

    def fill_body(i, carry):
        for jj in range(DW // 16):
            ones_v[i, pl.ds(jj * 16, 16)] = jnp.ones((16,), jnp.float32)
        return carry

    lax.fori_loop(0, K, fill_body, 0)
    pltpu.sync_copy(dst2_hbm.at[pl.ds(wid * DEG_CPT, DEG_CPT)], dsts_v)
    plsc.subcore_barrier()

    pltpu.async_copy(ones_v, acc_sh.at[dsts_v.at[0]], ssem, add=True)

    def chunk_body(j, carry):
        pltpu.async_copy(ones_v, acc_sh.at[dsts_v.at[j + 1]], ssem, add=True)
        pltpu.make_async_copy(ones_v, acc_sh.at[dsts_v.at[0]], ssem).wait()
        return carry

    lax.fori_loop(0, DEG_CPT - 1, chunk_body, 0)
    pltpu.make_async_copy(ones_v, acc_sh.at[dsts_v.at[0]], ssem).wait()
    plsc.subcore_barrier()

    @pl.when(cid == 0)
    def _():
        pltpu.sync_copy(acc_sh.at[pl.ds(r0, ROWS_PT)],
                        out0_hbm.at[pl.ds(r0, ROWS_PT)])

    @pl.when(cid == 1)
    def _():
        pltpu.sync_copy(acc_sh.at[pl.ds(r0, ROWS_PT)],
                        out1_hbm.at[pl.ds(r0, ROWS_PT)])


def _make_prop(F, feature_split):
    cpt = CH // (NSUB if feature_split else NW)
    IB = 40
    nblk = cpt // IB

    @functools.partial(
        pl.kernel,
        out_type=[
            jax.ShapeDtypeStruct((NP, F), jnp.float32),
            jax.ShapeDtypeStruct((NP, F), jnp.float32),
        ],
        mesh=_sc_mesh(),
        scratch_types=[
            pltpu.VMEM((IB, K), jnp.int32),
            pltpu.VMEM((IB, K), jnp.int32),
            pltpu.VMEM((K, F), jnp.float32),
            pltpu.VMEM((K, F), jnp.float32),
            pltpu.VMEM_SHARED((NP, F), jnp.float32),
            pltpu.SemaphoreType.DMA,
            pltpu.SemaphoreType.DMA,
        ],
    )
    def prop(u0_hbm, u1_hbm, src2_hbm, dst2_hbm, out0_hbm, out1_hbm,
             srcs_v, dsts_v, rows0_v, rows1_v, acc_sh, gs0, gs1):
        cid = lax.axis_index("c")
        sid = lax.axis_index("s")

        def zero_body(i, carry):
            for jj in range(F // 16):
                rows0_v[i, pl.ds(jj * 16, 16)] = jnp.zeros((16,), jnp.float32)
            return carry

        lax.fori_loop(0, K, zero_body, 0)
        r0 = sid * ROWS_PT
        for t in range(ROWS_PT // K):
            pltpu.sync_copy(rows0_v, acc_sh.at[pl.ds(r0 + t * K, K)])

        if feature_split:
            c0 = sid * cpt
        else:
            c0 = (cid * NSUB + sid) * cpt
        plsc.subcore_barrier()

        def run_edges(u_hbm, use_ring):
            bufs = (rows0_v, rows1_v)
            sems = (gs0, gs1)
            for blk in range(nblk):
                pltpu.sync_copy(
                    src2_hbm.at[pl.ds(c0 + blk * IB, IB)], srcs_v)
                pltpu.sync_copy(
                    dst2_hbm.at[pl.ds(c0 + blk * IB, IB)], dsts_v)
                if not use_ring:
                    def chunk_body(j, carry):
                        pltpu.async_copy(
                            u_hbm.at[srcs_v.at[j]], rows0_v, gs0).wait()
                        pltpu.sync_copy(rows0_v, acc_sh.at[dsts_v.at[j]],
                                        add=True)
                        return carry

                    lax.fori_loop(0, IB, chunk_body, 0)
                    continue
                pltpu.async_copy(u_hbm.at[srcs_v.at[0]], rows0_v, gs0)
                pltpu.async_copy(u_hbm.at[srcs_v.at[1]], rows1_v, gs1)

                def pair_body(t, carry):
                    j = 2 * t
                    for b in range(2):
                        buf, sem = bufs[b], sems[b]
                        pltpu.make_async_copy(
                            u_hbm.at[srcs_v.at[0]], buf, sem).wait()
                        pltpu.sync_copy(buf, acc_sh.at[dsts_v.at[j + b]],
                                        add=True)

                        @pl.when(j + b + 2 < IB)
                        def _():
                            pltpu.async_copy(
                                u_hbm.at[srcs_v.at[j + b + 2]], buf, sem)
                    return carry

                lax.fori_loop(0, IB // 2, pair_body, 0)

        if feature_split:
            @pl.when(cid == 0)
            def _():
                run_edges(u0_hbm, use_ring=True)

            @pl.when(cid == 1)
            def _():
                run_edges(u1_hbm, use_ring=True)
        else:
            run_edges(u0_hbm, use_ring=False)

        plsc.subcore_barrier()

        @pl.when(cid == 0)
        def _():
            pltpu.sync_copy(acc_sh.at[pl.ds(r0, ROWS_PT)],
                            out0_hbm.at[pl.ds(r0, ROWS_PT)])

        @pl.when(cid == 1)
        def _():
            pltpu.sync_copy(acc_sh.at[pl.ds(r0, ROWS_PT)],
                            out1_hbm.at[pl.ds(r0, ROWS_PT)])

    return prop


_prop1 = _make_prop(DIN, feature_split=False)
_prop2 = _make_prop(F2 // 2, feature_split=True)
_prop3 = _make_prop(DIN, feature_split=False)


def _tc_scale_body(d0_ref, d1_ref, x_ref, dinv_ref, u1_ref):
    deg = d0_ref[:, :1] + d1_ref[:, :1] + 1.0
    dinv = lax.rsqrt(deg)
    dinv_ref[...] = dinv
    u1_ref[...] = x_ref[...] * dinv


def _tc_scale(d0, d1, x_pad):
    return pl.pallas_call(
        _tc_scale_body,
        out_shape=[
            jax.ShapeDtypeStruct((NP, 1), jnp.float32),
            jax.ShapeDtypeStruct((NP, DIN), jnp.float32),
        ],
    )(d0, d1, x_pad)


R1 = 2048


def _tc_mm1_body(s1a, s1b, u1, dinv, W1, b1, W2, o_a, o_b):
    a1 = (s1a[...] + s1b[...] + u1[...]) * dinv[...]
    h1 = jnp.maximum(
        jnp.dot(a1, W1[...], preferred_element_type=jnp.float32) + b1[...], 0.0)
    z2 = jnp.dot(h1, W2[...], preferred_element_type=jnp.float32)
    u2 = z2 * dinv[...]
    o_a[...] = u2[:, :F2 // 2]
    o_b[...] = u2[:, F2 // 2:]


def _tc_mm1(s1a, s1b, u1, dinv, W1, b1, W2):
    nb = NP // R1
    return pl.pallas_call(
        _tc_mm1_body,
        grid=(nb,),
        in_specs=[
            pl.BlockSpec((R1, DIN), lambda i: (i, 0)),
            pl.BlockSpec((R1, DIN), lambda i: (i, 0)),
            pl.BlockSpec((R1, DIN), lambda i: (i, 0)),
            pl.BlockSpec((R1, 1), lambda i: (i, 0)),
            pl.BlockSpec((DIN, F1), lambda i: (0, 0)),
            pl.BlockSpec((1, F1), lambda i: (0, 0)),
            pl.BlockSpec((F1, F2), lambda i: (0, 0)),
        ],
        out_specs=[
            pl.BlockSpec((R1, F2 // 2), lambda i: (i, 0)),
            pl.BlockSpec((R1, F2 // 2), lambda i: (i, 0)),
        ],
        out_shape=[
            jax.ShapeDtypeStruct((NP, F2 // 2), jnp.float32),
            jax.ShapeDtypeStruct((NP, F2 // 2), jnp.float32),
        ],
    )(s1a, s1b, u1, dinv, W1, b1, W2)


def _tc_mm3_body(s2a, s2b, u2a, u2b, dinv, b2, W3, u3_ref):
    t = jnp.concatenate([s2a[...] + u2a[...], s2b[...] + u2b[...]], axis=1)
    h2 = jnp.maximum(t * dinv[...] + b2[...], 0.0)
    z3 = jnp.dot(h2, W3[...], preferred_element_type=jnp.float32)
    u3 = z3 * dinv[...]
    u3_ref[...] = jnp.concatenate(
        [u3, jnp.zeros((u3.shape[0], DIN - F3), jnp.float32)], axis=1)


def _tc_mm3(s2a, s2b, u2a, u2b, dinv, b2, W3):
    nb = NP // R1
    return pl.pallas_call(
        _tc_mm3_body,
        grid=(nb,),
        in_specs=[
            pl.BlockSpec((R1, F2 // 2), lambda i: (i, 0)),
            pl.BlockSpec((R1, F2 // 2), lambda i: (i, 0)),
            pl.BlockSpec((R1, F2 // 2), lambda i: (i, 0)),
            pl.BlockSpec((R1, F2 // 2), lambda i: (i, 0)),
            pl.BlockSpec((R1, 1), lambda i: (i, 0)),
            pl.BlockSpec((1, F2), lambda i: (0, 0)),
            pl.BlockSpec((F2, F3), lambda i: (0, 0)),
        ],
        out_specs=pl.BlockSpec((R1, DIN), lambda i: (i, 0)),
        out_shape=jax.ShapeDtypeStruct((NP, DIN), jnp.float32),
    )(s2a, s2b, u2a, u2b, dinv, b2, W3)


def _tc_head_body(s3a, s3b, u3, dinv, b3, batch_row, Wl, bl, out_ref):
    a3 = ((s3a[...] + s3b[...] + u3[...]) * dinv[...])[:, :F3] + b3[...]
    gids = lax.broadcasted_iota(jnp.int32, (G, NP), 0)
    oneh = (batch_row[...] == gids).astype(jnp.float32)
    sums = jnp.dot(oneh, a3, preferred_element_type=jnp.float32)
    cnt = jnp.dot(oneh, jnp.ones((NP, 1), jnp.float32),
                  preferred_element_type=jnp.float32)
    pooled = sums / jnp.maximum(cnt, 1.0)
    out_ref[...] = (
        jnp.dot(pooled, Wl[...], preferred_element_type=jnp.float32) + bl[...])


def _tc_head(s3a, s3b, u3, dinv, b3, batch_row, Wl, bl):
    return pl.pallas_call(
        _tc_head_body,
        out_shape=jax.ShapeDtypeStruct((G, NC), jnp.float32),
    )(s3a, s3b, u3, dinv, b3, batch_row, Wl, bl)


def kernel(x, edge_index, batch, W1, b1, W2, b2, W3, b3, Wl, bl):
    npad = EP - E
    src2 = jnp.concatenate(
        [edge_index[0], jnp.zeros((npad,), jnp.int32)]).reshape(CH, K)
    dst2 = jnp.concatenate(
        [edge_index[1], jnp.full((npad,), NP - 1, jnp.int32)]).reshape(CH, K)
    x_pad = jnp.pad(x, ((0, NP - N), (0, 0)))
    batch_row = jnp.pad(batch, (0, NP - N), constant_values=G)[None, :]

    d0, d1 = _deg_kernel(dst2)
    dinv, u1 = _tc_scale(d0, d1, x_pad)

    s1a, s1b = _prop1(u1, u1, src2, dst2)
    u2a, u2b = _tc_mm1(s1a, s1b, u1, dinv, W1, b1[None, :], W2)

    s2a, s2b = _prop2(u2a, u2b, src2, dst2)
    u3 = _tc_mm3(s2a, s2b, u2a, u2b, dinv, b2[None, :], W3)

    s3a, s3b = _prop3(u3, u3, src2, dst2)
    return _tc_head(s3a, s3b, u3, dinv, b3[None, :], batch_row, Wl, bl)


# speedup vs baseline: 1.0550x; 1.0550x over previous
import functools

import jax
import jax.numpy as jnp
from jax import lax
from jax.experimental import pallas as pl
from jax.experimental.pallas import tpu as pltpu
from jax.experimental.pallas import tpu_sc as plsc

N = 10000
NP = 10240
E = 320000
DIN = 128
F1 = 512
F2 = 256
F3 = 64
G = 64
NC = 10

NCORES = 2
NSUB = 16
NW = NCORES * NSUB
K = 128
CH = 2560
SPLIT0 = 0.25
EP = CH * K
ROWS_PT = NP // NSUB


def _sc_mesh():
    return plsc.VectorSubcoreMesh(core_axis_name="c", subcore_axis_name="s")


DW = 128


DEG_CPT = CH // NW


@functools.partial(
    pl.kernel,
    out_type=[
        jax.ShapeDtypeStruct((NP, DW), jnp.float32),
        jax.ShapeDtypeStruct((NP, DW), jnp.float32),
    ],
    mesh=_sc_mesh(),
    scratch_types=[
        pltpu.VMEM((DEG_CPT, K), jnp.int32),
        pltpu.VMEM((K, DW), jnp.float32),
        pltpu.VMEM_SHARED((NP, DW), jnp.float32),
        pltpu.SemaphoreType.DMA,
    ],
)
def _deg_kernel(dst2_hbm, out0_hbm, out1_hbm, dsts_v, ones_v, acc_sh, ssem):
    cid = lax.axis_index("c")
    sid = lax.axis_index("s")
    wid = cid * NSUB + sid

    def zero_body(i, carry):
        for jj in range(DW // 16):
            ones_v[i, pl.ds(jj * 16, 16)] = jnp.zeros((16,), jnp.float32)
        return carry

    lax.fori_loop(0, K, zero_body, 0)
    r0 = sid * ROWS_PT
    for t in range(ROWS_PT // K):
        pltpu.sync_copy(ones_v, acc_sh.at[pl.ds(r0 + t * K, K)])

    def fill_body(i, carry):
        for jj in range(DW // 16):
            ones_v[i, pl.ds(jj * 16, 16)] = jnp.ones((16,), jnp.float32)
        return carry

    lax.fori_loop(0, K, fill_body, 0)
    pltpu.sync_copy(dst2_hbm.at[pl.ds(wid * DEG_CPT, DEG_CPT)], dsts_v)
    plsc.subcore_barrier()

    pltpu.async_copy(ones_v, acc_sh.at[dsts_v.at[0]], ssem, add=True)

    def chunk_body(j, carry):
        pltpu.async_copy(ones_v, acc_sh.at[dsts_v.at[j + 1]], ssem, add=True)
        pltpu.make_async_copy(ones_v, acc_sh.at[dsts_v.at[0]], ssem).wait()
        return carry

    lax.fori_loop(0, DEG_CPT - 1, chunk_body, 0)
    pltpu.make_async_copy(ones_v, acc_sh.at[dsts_v.at[0]], ssem).wait()
    plsc.subcore_barrier()

    @pl.when(cid == 0)
    def _():
        pltpu.sync_copy(acc_sh.at[pl.ds(r0, ROWS_PT)],
                        out0_hbm.at[pl.ds(r0, ROWS_PT)])

    @pl.when(cid == 1)
    def _():
        pltpu.sync_copy(acc_sh.at[pl.ds(r0, ROWS_PT)],
                        out1_hbm.at[pl.ds(r0, ROWS_PT)])


def _make_prop(F, feature_split):
    if feature_split:
        cpt0 = cpt1 = CH // NSUB
    else:
        ch0 = int(CH * SPLIT0) // 128 * 128
        cpt0 = ch0 // NSUB
        cpt1 = (CH - ch0) // NSUB
    IB = 40
    assert cpt0 % IB == 0 and cpt1 % IB == 0

    @functools.partial(
        pl.kernel,
        out_type=[
            jax.ShapeDtypeStruct((NP, F), jnp.float32),
            jax.ShapeDtypeStruct((NP, F), jnp.float32),
        ],
        mesh=_sc_mesh(),
        scratch_types=[
            pltpu.VMEM((IB, K), jnp.int32),
            pltpu.VMEM((IB, K), jnp.int32),
            pltpu.VMEM((K, F), jnp.float32),
            pltpu.VMEM((K, F), jnp.float32),
            pltpu.VMEM_SHARED((NP, F), jnp.float32),
            pltpu.SemaphoreType.DMA,
            pltpu.SemaphoreType.DMA,
        ],
    )
    def prop(u0_hbm, u1_hbm, src2_hbm, dst2_hbm, out0_hbm, out1_hbm,
             srcs_v, dsts_v, rows0_v, rows1_v, acc_sh, gs0, gs1):
        cid = lax.axis_index("c")
        sid = lax.axis_index("s")

        def zero_body(i, carry):
            for jj in range(F // 16):
                rows0_v[i, pl.ds(jj * 16, 16)] = jnp.zeros((16,), jnp.float32)
            return carry

        lax.fori_loop(0, K, zero_body, 0)
        r0 = sid * ROWS_PT
        for t in range(ROWS_PT // K):
            pltpu.sync_copy(rows0_v, acc_sh.at[pl.ds(r0 + t * K, K)])

        plsc.subcore_barrier()

        def run_edges(u_hbm, c0, cpt):
            bufs = (rows0_v, rows1_v)
            sems = (gs0, gs1)
            for blk in range(cpt // IB):
                pltpu.sync_copy(
                    src2_hbm.at[pl.ds(c0 + blk * IB, IB)], srcs_v)
                pltpu.sync_copy(
                    dst2_hbm.at[pl.ds(c0 + blk * IB, IB)], dsts_v)
                pltpu.async_copy(u_hbm.at[srcs_v.at[0]], rows0_v, gs0)
                pltpu.async_copy(u_hbm.at[srcs_v.at[1]], rows1_v, gs1)

                def pair_body(t, carry):
                    j = 2 * t
                    for b in range(2):
                        buf, sem = bufs[b], sems[b]
                        pltpu.make_async_copy(
                            u_hbm.at[srcs_v.at[0]], buf, sem).wait()
                        pltpu.sync_copy(buf, acc_sh.at[dsts_v.at[j + b]],
                                        add=True)

                        @pl.when(j + b + 2 < IB)
                        def _():
                            pltpu.async_copy(
                                u_hbm.at[srcs_v.at[j + b + 2]], buf, sem)
                    return carry

                lax.fori_loop(0, IB // 2, pair_body, 0)

        if feature_split:
            @pl.when(cid == 0)
            def _():
                run_edges(u0_hbm, sid * cpt0, cpt0)

            @pl.when(cid == 1)
            def _():
                run_edges(u1_hbm, sid * cpt1, cpt1)
        else:
            @pl.when(cid == 0)
            def _():
                run_edges(u0_hbm, sid * cpt0, cpt0)

            @pl.when(cid == 1)
            def _():
                run_edges(u0_hbm, cpt0 * NSUB + sid * cpt1, cpt1)

        plsc.subcore_barrier()

        @pl.when(cid == 0)
        def _():
            pltpu.sync_copy(acc_sh.at[pl.ds(r0, ROWS_PT)],
                            out0_hbm.at[pl.ds(r0, ROWS_PT)])

        @pl.when(cid == 1)
        def _():
            pltpu.sync_copy(acc_sh.at[pl.ds(r0, ROWS_PT)],
                            out1_hbm.at[pl.ds(r0, ROWS_PT)])

    return prop


_prop1 = _make_prop(DIN, feature_split=False)
_prop2 = _make_prop(F2 // 2, feature_split=True)
_prop3 = _make_prop(DIN, feature_split=False)


def _tc_scale_body(d0_ref, d1_ref, x_ref, dinv_ref, u1_ref):
    deg = d0_ref[:, :1] + d1_ref[:, :1] + 1.0
    dinv = lax.rsqrt(deg)
    dinv_ref[...] = dinv
    u1_ref[...] = x_ref[...] * dinv


def _tc_scale(d0, d1, x_pad):
    return pl.pallas_call(
        _tc_scale_body,
        out_shape=[
            jax.ShapeDtypeStruct((NP, 1), jnp.float32),
            jax.ShapeDtypeStruct((NP, DIN), jnp.float32),
        ],
    )(d0, d1, x_pad)


R1 = 2048


def _tc_mm1_body(s1a, s1b, u1, dinv, W1, b1, W2, o_a, o_b):
    a1 = (s1a[...] + s1b[...] + u1[...]) * dinv[...]
    h1 = jnp.maximum(
        jnp.dot(a1, W1[...], preferred_element_type=jnp.float32) + b1[...], 0.0)
    z2 = jnp.dot(h1, W2[...], preferred_element_type=jnp.float32)
    u2 = z2 * dinv[...]
    o_a[...] = u2[:, :F2 // 2]
    o_b[...] = u2[:, F2 // 2:]


def _tc_mm1(s1a, s1b, u1, dinv, W1, b1, W2):
    nb = NP // R1
    return pl.pallas_call(
        _tc_mm1_body,
        grid=(nb,),
        in_specs=[
            pl.BlockSpec((R1, DIN), lambda i: (i, 0)),
            pl.BlockSpec((R1, DIN), lambda i: (i, 0)),
            pl.BlockSpec((R1, DIN), lambda i: (i, 0)),
            pl.BlockSpec((R1, 1), lambda i: (i, 0)),
            pl.BlockSpec((DIN, F1), lambda i: (0, 0)),
            pl.BlockSpec((1, F1), lambda i: (0, 0)),
            pl.BlockSpec((F1, F2), lambda i: (0, 0)),
        ],
        out_specs=[
            pl.BlockSpec((R1, F2 // 2), lambda i: (i, 0)),
            pl.BlockSpec((R1, F2 // 2), lambda i: (i, 0)),
        ],
        out_shape=[
            jax.ShapeDtypeStruct((NP, F2 // 2), jnp.float32),
            jax.ShapeDtypeStruct((NP, F2 // 2), jnp.float32),
        ],
    )(s1a, s1b, u1, dinv, W1, b1, W2)


def _tc_mm3_body(s2a, s2b, u2a, u2b, dinv, b2, W3, u3_ref):
    t = jnp.concatenate([s2a[...] + u2a[...], s2b[...] + u2b[...]], axis=1)
    h2 = jnp.maximum(t * dinv[...] + b2[...], 0.0)
    z3 = jnp.dot(h2, W3[...], preferred_element_type=jnp.float32)
    u3 = z3 * dinv[...]
    u3_ref[...] = jnp.concatenate(
        [u3, jnp.zeros((u3.shape[0], DIN - F3), jnp.float32)], axis=1)


def _tc_mm3(s2a, s2b, u2a, u2b, dinv, b2, W3):
    nb = NP // R1
    return pl.pallas_call(
        _tc_mm3_body,
        grid=(nb,),
        in_specs=[
            pl.BlockSpec((R1, F2 // 2), lambda i: (i, 0)),
            pl.BlockSpec((R1, F2 // 2), lambda i: (i, 0)),
            pl.BlockSpec((R1, F2 // 2), lambda i: (i, 0)),
            pl.BlockSpec((R1, F2 // 2), lambda i: (i, 0)),
            pl.BlockSpec((R1, 1), lambda i: (i, 0)),
            pl.BlockSpec((1, F2), lambda i: (0, 0)),
            pl.BlockSpec((F2, F3), lambda i: (0, 0)),
        ],
        out_specs=pl.BlockSpec((R1, DIN), lambda i: (i, 0)),
        out_shape=jax.ShapeDtypeStruct((NP, DIN), jnp.float32),
    )(s2a, s2b, u2a, u2b, dinv, b2, W3)


def _tc_head_body(s3a, s3b, u3, dinv, b3, batch_row, Wl, bl, out_ref):
    a3 = ((s3a[...] + s3b[...] + u3[...]) * dinv[...])[:, :F3] + b3[...]
    gids = lax.broadcasted_iota(jnp.int32, (G, NP), 0)
    oneh = (batch_row[...] == gids).astype(jnp.float32)
    sums = jnp.dot(oneh, a3, preferred_element_type=jnp.float32)
    cnt = jnp.dot(oneh, jnp.ones((NP, 1), jnp.float32),
                  preferred_element_type=jnp.float32)
    pooled = sums / jnp.maximum(cnt, 1.0)
    out_ref[...] = (
        jnp.dot(pooled, Wl[...], preferred_element_type=jnp.float32) + bl[...])


def _tc_head(s3a, s3b, u3, dinv, b3, batch_row, Wl, bl):
    return pl.pallas_call(
        _tc_head_body,
        out_shape=jax.ShapeDtypeStruct((G, NC), jnp.float32),
    )(s3a, s3b, u3, dinv, b3, batch_row, Wl, bl)


def kernel(x, edge_index, batch, W1, b1, W2, b2, W3, b3, Wl, bl):
    npad = EP - E
    src2 = jnp.concatenate(
        [edge_index[0], jnp.zeros((npad,), jnp.int32)]).reshape(CH, K)
    dst2 = jnp.concatenate(
        [edge_index[1], jnp.full((npad,), NP - 1, jnp.int32)]).reshape(CH, K)
    x_pad = jnp.pad(x, ((0, NP - N), (0, 0)))
    batch_row = jnp.pad(batch, (0, NP - N), constant_values=G)[None, :]

    d0, d1 = _deg_kernel(dst2)
    dinv, u1 = _tc_scale(d0, d1, x_pad)

    s1a, s1b = _prop1(u1, u1, src2, dst2)
    u2a, u2b = _tc_mm1(s1a, s1b, u1, dinv, W1, b1[None, :], W2)

    s2a, s2b = _prop2(u2a, u2b, src2, dst2)
    u3 = _tc_mm3(s2a, s2b, u2a, u2b, dinv, b2[None, :], W3)

    s3a, s3b = _prop3(u3, u3, src2, dst2)
    return _tc_head(s3a, s3b, u3, dinv, b3[None, :], batch_row, Wl, bl)


# speedup vs baseline: 1.1080x; 1.0503x over previous
import functools

import jax
import jax.numpy as jnp
from jax import lax
from jax.experimental import pallas as pl
from jax.experimental.pallas import tpu as pltpu
from jax.experimental.pallas import tpu_sc as plsc

N = 10000
NP = 10240
E = 320000
DIN = 128
F1 = 512
F2 = 256
F3 = 64
G = 64
NC = 10

NCORES = 2
NSUB = 16
NW = NCORES * NSUB
K = 128
CH = 2560
SPLIT0 = 0.75
EP = CH * K
ROWS_PT = NP // NSUB


def _sc_mesh():
    return plsc.VectorSubcoreMesh(core_axis_name="c", subcore_axis_name="s")


DW = 128


DEG_CPT = CH // NW


@functools.partial(
    pl.kernel,
    out_type=[
        jax.ShapeDtypeStruct((NP, DW), jnp.float32),
        jax.ShapeDtypeStruct((NP, DW), jnp.float32),
    ],
    mesh=_sc_mesh(),
    scratch_types=[
        pltpu.VMEM((DEG_CPT, K), jnp.int32),
        pltpu.VMEM((K, DW), jnp.float32),
        pltpu.VMEM_SHARED((NP, DW), jnp.float32),
        pltpu.SemaphoreType.DMA,
    ],
)
def _deg_kernel(dst2_hbm, out0_hbm, out1_hbm, dsts_v, ones_v, acc_sh, ssem):
    cid = lax.axis_index("c")
    sid = lax.axis_index("s")
    wid = cid * NSUB + sid

    def zero_body(i, carry):
        for jj in range(DW // 16):
            ones_v[i, pl.ds(jj * 16, 16)] = jnp.zeros((16,), jnp.float32)
        return carry

    lax.fori_loop(0, K, zero_body, 0)
    r0 = sid * ROWS_PT
    for t in range(ROWS_PT // K):
        pltpu.sync_copy(ones_v, acc_sh.at[pl.ds(r0 + t * K, K)])

    def fill_body(i, carry):
        for jj in range(DW // 16):
            ones_v[i, pl.ds(jj * 16, 16)] = jnp.ones((16,), jnp.float32)
        return carry

    lax.fori_loop(0, K, fill_body, 0)
    pltpu.sync_copy(dst2_hbm.at[pl.ds(wid * DEG_CPT, DEG_CPT)], dsts_v)
    plsc.subcore_barrier()

    pltpu.async_copy(ones_v, acc_sh.at[dsts_v.at[0]], ssem, add=True)

    def chunk_body(j, carry):
        pltpu.async_copy(ones_v, acc_sh.at[dsts_v.at[j + 1]], ssem, add=True)
        pltpu.make_async_copy(ones_v, acc_sh.at[dsts_v.at[0]], ssem).wait()
        return carry

    lax.fori_loop(0, DEG_CPT - 1, chunk_body, 0)
    pltpu.make_async_copy(ones_v, acc_sh.at[dsts_v.at[0]], ssem).wait()
    plsc.subcore_barrier()

    @pl.when(cid == 0)
    def _():
        pltpu.sync_copy(acc_sh.at[pl.ds(r0, ROWS_PT)],
                        out0_hbm.at[pl.ds(r0, ROWS_PT)])

    @pl.when(cid == 1)
    def _():
        pltpu.sync_copy(acc_sh.at[pl.ds(r0, ROWS_PT)],
                        out1_hbm.at[pl.ds(r0, ROWS_PT)])


def _make_prop(F, feature_split):
    if feature_split:
        cpt0 = cpt1 = CH // NSUB
    else:
        ch0 = int(CH * SPLIT0) // 128 * 128
        cpt0 = ch0 // NSUB
        cpt1 = (CH - ch0) // NSUB
    IB = 40
    assert cpt0 % IB == 0 and cpt1 % IB == 0

    @functools.partial(
        pl.kernel,
        out_type=[
            jax.ShapeDtypeStruct((NP, F), jnp.float32),
            jax.ShapeDtypeStruct((NP, F), jnp.float32),
        ],
        mesh=_sc_mesh(),
        scratch_types=[
            pltpu.VMEM((IB, K), jnp.int32),
            pltpu.VMEM((IB, K), jnp.int32),
            pltpu.VMEM((K, F), jnp.float32),
            pltpu.VMEM((K, F), jnp.float32),
            pltpu.VMEM_SHARED((NP, F), jnp.float32),
            pltpu.SemaphoreType.DMA,
            pltpu.SemaphoreType.DMA,
        ],
    )
    def prop(u0_hbm, u1_hbm, src2_hbm, dst2_hbm, out0_hbm, out1_hbm,
             srcs_v, dsts_v, rows0_v, rows1_v, acc_sh, gs0, gs1):
        cid = lax.axis_index("c")
        sid = lax.axis_index("s")

        def zero_body(i, carry):
            for jj in range(F // 16):
                rows0_v[i, pl.ds(jj * 16, 16)] = jnp.zeros((16,), jnp.float32)
            return carry

        lax.fori_loop(0, K, zero_body, 0)
        r0 = sid * ROWS_PT
        for t in range(ROWS_PT // K):
            pltpu.sync_copy(rows0_v, acc_sh.at[pl.ds(r0 + t * K, K)])

        plsc.subcore_barrier()

        def run_edges(u_hbm, c0, cpt):
            bufs = (rows0_v, rows1_v)
            sems = (gs0, gs1)
            for blk in range(cpt // IB):
                pltpu.sync_copy(
                    src2_hbm.at[pl.ds(c0 + blk * IB, IB)], srcs_v)
                pltpu.sync_copy(
                    dst2_hbm.at[pl.ds(c0 + blk * IB, IB)], dsts_v)
                pltpu.async_copy(u_hbm.at[srcs_v.at[0]], rows0_v, gs0)
                pltpu.async_copy(u_hbm.at[srcs_v.at[1]], rows1_v, gs1)

                def pair_body(t, carry):
                    j = 2 * t
                    for b in range(2):
                        buf, sem = bufs[b], sems[b]
                        pltpu.make_async_copy(
                            u_hbm.at[srcs_v.at[0]], buf, sem).wait()
                        pltpu.sync_copy(buf, acc_sh.at[dsts_v.at[j + b]],
                                        add=True)

                        @pl.when(j + b + 2 < IB)
                        def _():
                            pltpu.async_copy(
                                u_hbm.at[srcs_v.at[j + b + 2]], buf, sem)
                    return carry

                lax.fori_loop(0, IB // 2, pair_body, 0)

        if feature_split:
            @pl.when(cid == 0)
            def _():
                run_edges(u0_hbm, sid * cpt0, cpt0)

            @pl.when(cid == 1)
            def _():
                run_edges(u1_hbm, sid * cpt1, cpt1)
        else:
            @pl.when(cid == 0)
            def _():
                run_edges(u0_hbm, sid * cpt0, cpt0)

            @pl.when(cid == 1)
            def _():
                run_edges(u0_hbm, cpt0 * NSUB + sid * cpt1, cpt1)

        plsc.subcore_barrier()

        @pl.when(cid == 0)
        def _():
            pltpu.sync_copy(acc_sh.at[pl.ds(r0, ROWS_PT)],
                            out0_hbm.at[pl.ds(r0, ROWS_PT)])

        @pl.when(cid == 1)
        def _():
            pltpu.sync_copy(acc_sh.at[pl.ds(r0, ROWS_PT)],
                            out1_hbm.at[pl.ds(r0, ROWS_PT)])

    return prop


_prop1 = _make_prop(DIN, feature_split=False)
_prop2 = _make_prop(F2 // 2, feature_split=True)
_prop3 = _make_prop(DIN, feature_split=False)


def _tc_scale_body(d0_ref, d1_ref, x_ref, dinv_ref, u1_ref):
    deg = d0_ref[:, :1] + d1_ref[:, :1] + 1.0
    dinv = lax.rsqrt(deg)
    dinv_ref[...] = dinv
    u1_ref[...] = x_ref[...] * dinv


def _tc_scale(d0, d1, x_pad):
    return pl.pallas_call(
        _tc_scale_body,
        out_shape=[
            jax.ShapeDtypeStruct((NP, 1), jnp.float32),
            jax.ShapeDtypeStruct((NP, DIN), jnp.float32),
        ],
    )(d0, d1, x_pad)


R1 = 2048


def _tc_mm1_body(s1a, s1b, u1, dinv, W1, b1, W2, o_a, o_b):
    a1 = (s1a[...] + s1b[...] + u1[...]) * dinv[...]
    h1 = jnp.maximum(
        jnp.dot(a1, W1[...], preferred_element_type=jnp.float32) + b1[...], 0.0)
    z2 = jnp.dot(h1, W2[...], preferred_element_type=jnp.float32)
    u2 = z2 * dinv[...]
    o_a[...] = u2[:, :F2 // 2]
    o_b[...] = u2[:, F2 // 2:]


def _tc_mm1(s1a, s1b, u1, dinv, W1, b1, W2):
    nb = NP // R1
    return pl.pallas_call(
        _tc_mm1_body,
        grid=(nb,),
        in_specs=[
            pl.BlockSpec((R1, DIN), lambda i: (i, 0)),
            pl.BlockSpec((R1, DIN), lambda i: (i, 0)),
            pl.BlockSpec((R1, DIN), lambda i: (i, 0)),
            pl.BlockSpec((R1, 1), lambda i: (i, 0)),
            pl.BlockSpec((DIN, F1), lambda i: (0, 0)),
            pl.BlockSpec((1, F1), lambda i: (0, 0)),
            pl.BlockSpec((F1, F2), lambda i: (0, 0)),
        ],
        out_specs=[
            pl.BlockSpec((R1, F2 // 2), lambda i: (i, 0)),
            pl.BlockSpec((R1, F2 // 2), lambda i: (i, 0)),
        ],
        out_shape=[
            jax.ShapeDtypeStruct((NP, F2 // 2), jnp.float32),
            jax.ShapeDtypeStruct((NP, F2 // 2), jnp.float32),
        ],
    )(s1a, s1b, u1, dinv, W1, b1, W2)


def _tc_mm3_body(s2a, s2b, u2a, u2b, dinv, b2, W3, u3_ref):
    t = jnp.concatenate([s2a[...] + u2a[...], s2b[...] + u2b[...]], axis=1)
    h2 = jnp.maximum(t * dinv[...] + b2[...], 0.0)
    z3 = jnp.dot(h2, W3[...], preferred_element_type=jnp.float32)
    u3 = z3 * dinv[...]
    u3_ref[...] = jnp.concatenate(
        [u3, jnp.zeros((u3.shape[0], DIN - F3), jnp.float32)], axis=1)


def _tc_mm3(s2a, s2b, u2a, u2b, dinv, b2, W3):
    nb = NP // R1
    return pl.pallas_call(
        _tc_mm3_body,
        grid=(nb,),
        in_specs=[
            pl.BlockSpec((R1, F2 // 2), lambda i: (i, 0)),
            pl.BlockSpec((R1, F2 // 2), lambda i: (i, 0)),
            pl.BlockSpec((R1, F2 // 2), lambda i: (i, 0)),
            pl.BlockSpec((R1, F2 // 2), lambda i: (i, 0)),
            pl.BlockSpec((R1, 1), lambda i: (i, 0)),
            pl.BlockSpec((1, F2), lambda i: (0, 0)),
            pl.BlockSpec((F2, F3), lambda i: (0, 0)),
        ],
        out_specs=pl.BlockSpec((R1, DIN), lambda i: (i, 0)),
        out_shape=jax.ShapeDtypeStruct((NP, DIN), jnp.float32),
    )(s2a, s2b, u2a, u2b, dinv, b2, W3)


def _tc_head_body(s3a, s3b, u3, dinv, b3, batch_row, Wl, bl, out_ref):
    a3 = ((s3a[...] + s3b[...] + u3[...]) * dinv[...])[:, :F3] + b3[...]
    gids = lax.broadcasted_iota(jnp.int32, (G, NP), 0)
    oneh = (batch_row[...] == gids).astype(jnp.float32)
    sums = jnp.dot(oneh, a3, preferred_element_type=jnp.float32)
    cnt = jnp.dot(oneh, jnp.ones((NP, 1), jnp.float32),
                  preferred_element_type=jnp.float32)
    pooled = sums / jnp.maximum(cnt, 1.0)
    out_ref[...] = (
        jnp.dot(pooled, Wl[...], preferred_element_type=jnp.float32) + bl[...])


def _tc_head(s3a, s3b, u3, dinv, b3, batch_row, Wl, bl):
    return pl.pallas_call(
        _tc_head_body,
        out_shape=jax.ShapeDtypeStruct((G, NC), jnp.float32),
    )(s3a, s3b, u3, dinv, b3, batch_row, Wl, bl)


def kernel(x, edge_index, batch, W1, b1, W2, b2, W3, b3, Wl, bl):
    npad = EP - E
    src2 = jnp.concatenate(
        [edge_index[0], jnp.zeros((npad,), jnp.int32)]).reshape(CH, K)
    dst2 = jnp.concatenate(
        [edge_index[1], jnp.full((npad,), NP - 1, jnp.int32)]).reshape(CH, K)
    x_pad = jnp.pad(x, ((0, NP - N), (0, 0)))
    batch_row = jnp.pad(batch, (0, NP - N), constant_values=G)[None, :]

    d0, d1 = _deg_kernel(dst2)
    dinv, u1 = _tc_scale(d0, d1, x_pad)

    s1a, s1b = _prop1(u1, u1, src2, dst2)
    u2a, u2b = _tc_mm1(s1a, s1b, u1, dinv, W1, b1[None, :], W2)

    s2a, s2b = _prop2(u2a, u2b, src2, dst2)
    u3 = _tc_mm3(s2a, s2b, u2a, u2b, dinv, b2[None, :], W3)

    s3a, s3b = _prop3(u3, u3, src2, dst2)
    return _tc_head(s3a, s3b, u3, dinv, b3[None, :], batch_row, Wl, bl)


# speedup vs baseline: 1.1879x; 1.0720x over previous
import functools

import jax
import jax.numpy as jnp
from jax import lax
from jax.experimental import pallas as pl
from jax.experimental.pallas import tpu as pltpu
from jax.experimental.pallas import tpu_sc as plsc

N = 10000
NP = 10240
E = 320000
DIN = 128
F1 = 512
F2 = 256
F3 = 64
G = 64
NC = 10

NCORES = 2
NSUB = 16
NW = NCORES * NSUB
K = 128
CH = 2560
SPLIT0 = 0.75
EP = CH * K
ROWS_PT = NP // NSUB


def _sc_mesh():
    return plsc.VectorSubcoreMesh(core_axis_name="c", subcore_axis_name="s")


DW = 128


DEG_CPT = CH // NW


@functools.partial(
    pl.kernel,
    out_type=[
        jax.ShapeDtypeStruct((NP, DW), jnp.float32),
        jax.ShapeDtypeStruct((NP, DW), jnp.float32),
    ],
    mesh=_sc_mesh(),
    scratch_types=[
        pltpu.VMEM((DEG_CPT, K), jnp.int32),
        pltpu.VMEM((K, DW), jnp.float32),
        pltpu.VMEM_SHARED((NP, DW), jnp.float32),
        pltpu.SemaphoreType.DMA,
    ],
)
def _deg_kernel(dst2_hbm, out0_hbm, out1_hbm, dsts_v, ones_v, acc_sh, ssem):
    cid = lax.axis_index("c")
    sid = lax.axis_index("s")
    wid = cid * NSUB + sid

    def zero_body(i, carry):
        for jj in range(DW // 16):
            ones_v[i, pl.ds(jj * 16, 16)] = jnp.zeros((16,), jnp.float32)
        return carry

    lax.fori_loop(0, K, zero_body, 0)
    r0 = sid * ROWS_PT
    for t in range(ROWS_PT // K):
        pltpu.sync_copy(ones_v, acc_sh.at[pl.ds(r0 + t * K, K)])

    def fill_body(i, carry):
        for jj in range(DW // 16):
            ones_v[i, pl.ds(jj * 16, 16)] = jnp.ones((16,), jnp.float32)
        return carry

    lax.fori_loop(0, K, fill_body, 0)
    pltpu.sync_copy(dst2_hbm.at[pl.ds(wid * DEG_CPT, DEG_CPT)], dsts_v)
    plsc.subcore_barrier()

    pltpu.async_copy(ones_v, acc_sh.at[dsts_v.at[0]], ssem, add=True)

    def chunk_body(j, carry):
        pltpu.async_copy(ones_v, acc_sh.at[dsts_v.at[j + 1]], ssem, add=True)
        pltpu.make_async_copy(ones_v, acc_sh.at[dsts_v.at[0]], ssem).wait()
        return carry

    lax.fori_loop(0, DEG_CPT - 1, chunk_body, 0)
    pltpu.make_async_copy(ones_v, acc_sh.at[dsts_v.at[0]], ssem).wait()
    plsc.subcore_barrier()

    @pl.when(cid == 0)
    def _():
        pltpu.sync_copy(acc_sh.at[pl.ds(r0, ROWS_PT)],
                        out0_hbm.at[pl.ds(r0, ROWS_PT)])

    @pl.when(cid == 1)
    def _():
        pltpu.sync_copy(acc_sh.at[pl.ds(r0, ROWS_PT)],
                        out1_hbm.at[pl.ds(r0, ROWS_PT)])


def _make_prop(F, feature_split):
    if feature_split:
        cpt0 = cpt1 = CH // NSUB
    else:
        ch0 = int(CH * SPLIT0) // 128 * 128
        cpt0 = ch0 // NSUB
        cpt1 = (CH - ch0) // NSUB
    IB = 40
    assert cpt0 % IB == 0 and cpt1 % IB == 0

    @functools.partial(
        pl.kernel,
        out_type=[
            jax.ShapeDtypeStruct((NP, F), jnp.float32),
            jax.ShapeDtypeStruct((NP, F), jnp.float32),
        ],
        mesh=_sc_mesh(),
        scratch_types=[
            pltpu.VMEM((IB, K), jnp.int32),
            pltpu.VMEM((IB, K), jnp.int32),
            pltpu.VMEM((K, F), jnp.float32),
            pltpu.VMEM((K, F), jnp.float32),
            pltpu.VMEM_SHARED((NP, F), jnp.float32),
            pltpu.SemaphoreType.DMA,
            pltpu.SemaphoreType.DMA,
        ],
    )
    def prop(u0_hbm, u1_hbm, src2_hbm, dst2_hbm, out0_hbm, out1_hbm,
             srcs_v, dsts_v, rows0_v, rows1_v, acc_sh, gs0, gs1):
        cid = lax.axis_index("c")
        sid = lax.axis_index("s")

        def zero_body(i, carry):
            for jj in range(F // 16):
                rows0_v[i, pl.ds(jj * 16, 16)] = jnp.zeros((16,), jnp.float32)
            return carry

        lax.fori_loop(0, K, zero_body, 0)
        r0 = sid * ROWS_PT
        for t in range(ROWS_PT // K):
            pltpu.sync_copy(rows0_v, acc_sh.at[pl.ds(r0 + t * K, K)])

        plsc.subcore_barrier()

        def run_edges(u_hbm, c0, cpt):
            bufs = (rows0_v, rows1_v)
            sems = (gs0, gs1)
            for blk in range(cpt // IB):
                pltpu.sync_copy(
                    src2_hbm.at[pl.ds(c0 + blk * IB, IB)], srcs_v)
                pltpu.sync_copy(
                    dst2_hbm.at[pl.ds(c0 + blk * IB, IB)], dsts_v)
                pltpu.async_copy(u_hbm.at[srcs_v.at[0]], rows0_v, gs0)
                pltpu.async_copy(u_hbm.at[srcs_v.at[1]], rows1_v, gs1)

                def pair_body(t, carry):
                    j = 2 * t
                    for b in range(2):
                        buf, sem = bufs[b], sems[b]
                        pltpu.make_async_copy(
                            u_hbm.at[srcs_v.at[0]], buf, sem).wait()
                        pltpu.sync_copy(buf, acc_sh.at[dsts_v.at[j + b]],
                                        add=True)

                        @pl.when(j + b + 2 < IB)
                        def _():
                            pltpu.async_copy(
                                u_hbm.at[srcs_v.at[j + b + 2]], buf, sem)
                    return carry

                lax.fori_loop(0, IB // 2, pair_body, 0)

        if feature_split:
            @pl.when(cid == 0)
            def _():
                run_edges(u0_hbm, sid * cpt0, cpt0)

            @pl.when(cid == 1)
            def _():
                run_edges(u1_hbm, sid * cpt1, cpt1)
        else:
            @pl.when(cid == 0)
            def _():
                run_edges(u0_hbm, sid * cpt0, cpt0)

            @pl.when(cid == 1)
            def _():
                run_edges(u0_hbm, cpt0 * NSUB + sid * cpt1, cpt1)

        plsc.subcore_barrier()

        @pl.when(cid == 0)
        def _():
            pltpu.sync_copy(acc_sh.at[pl.ds(r0, ROWS_PT)],
                            out0_hbm.at[pl.ds(r0, ROWS_PT)])

        @pl.when(cid == 1)
        def _():
            pltpu.sync_copy(acc_sh.at[pl.ds(r0, ROWS_PT)],
                            out1_hbm.at[pl.ds(r0, ROWS_PT)])

    return prop


_prop2 = _make_prop(F2 // 2, feature_split=True)


def _make_prop_sync(F):
    K8 = 80
    ept = E // NW
    nchunks = ept // K8

    @functools.partial(
        pl.kernel,
        out_type=[
            jax.ShapeDtypeStruct((NP, F), jnp.float32),
            jax.ShapeDtypeStruct((NP, F), jnp.float32),
        ],
        mesh=_sc_mesh(),
        scratch_types=[
            pltpu.VMEM((K8,), jnp.int32),
            pltpu.VMEM((K8,), jnp.int32),
            pltpu.VMEM((K8, F), jnp.float32),
            pltpu.VMEM_SHARED((NP, F), jnp.float32),
            pltpu.SemaphoreType.DMA,
        ],
    )
    def prop(u0_hbm, u1_hbm, src_hbm, dst_hbm, out0_hbm, out1_hbm,
             src_v, dst_v, rows_v, acc_sh, sem):
        cid = lax.axis_index("c")
        sid = lax.axis_index("s")

        def zero_body(i, carry):
            for jj in range(F // 16):
                rows_v[i, pl.ds(jj * 16, 16)] = jnp.zeros((16,), jnp.float32)
            return carry

        lax.fori_loop(0, K8, zero_body, 0)
        r0 = sid * ROWS_PT
        for t in range(ROWS_PT // K8):
            pltpu.sync_copy(rows_v, acc_sh.at[pl.ds(r0 + t * K8, K8)])
        plsc.subcore_barrier()

        tile_base = (cid * NSUB + sid) * ept

        def run_edges(u_hbm):
            def body(j, carry):
                base = tile_base + j * K8
                pltpu.sync_copy(src_hbm.at[pl.ds(base, K8)], src_v)
                pltpu.sync_copy(dst_hbm.at[pl.ds(base, K8)], dst_v)
                pltpu.async_copy(u_hbm.at[src_v], rows_v, sem).wait()
                pltpu.sync_copy(rows_v, acc_sh.at[dst_v], add=True)
                return carry

            lax.fori_loop(0, nchunks, body, 0)

        @pl.when(cid == 0)
        def _():
            run_edges(u0_hbm)

        @pl.when(cid == 1)
        def _():
            run_edges(u1_hbm)

        plsc.subcore_barrier()

        @pl.when(cid == 0)
        def _():
            pltpu.sync_copy(acc_sh.at[pl.ds(r0, ROWS_PT)],
                            out0_hbm.at[pl.ds(r0, ROWS_PT)])

        @pl.when(cid == 1)
        def _():
            pltpu.sync_copy(acc_sh.at[pl.ds(r0, ROWS_PT)],
                            out1_hbm.at[pl.ds(r0, ROWS_PT)])

    return prop


_prop13 = _make_prop_sync(DIN)
_prop1 = _prop13
_prop3 = _prop13


def _tc_scale_body(d0_ref, d1_ref, x_ref, dinv_ref, u1_ref):
    deg = d0_ref[:, :1] + d1_ref[:, :1] + 1.0
    dinv = lax.rsqrt(deg)
    dinv_ref[...] = dinv
    u1_ref[...] = x_ref[...] * dinv


def _tc_scale(d0, d1, x_pad):
    return pl.pallas_call(
        _tc_scale_body,
        out_shape=[
            jax.ShapeDtypeStruct((NP, 1), jnp.float32),
            jax.ShapeDtypeStruct((NP, DIN), jnp.float32),
        ],
    )(d0, d1, x_pad)


R1 = 2048


def _tc_mm1_body(s1a, s1b, u1, dinv, W1, b1, W2, o_a, o_b):
    a1 = (s1a[...] + s1b[...] + u1[...]) * dinv[...]
    h1 = jnp.maximum(
        jnp.dot(a1, W1[...], preferred_element_type=jnp.float32) + b1[...], 0.0)
    z2 = jnp.dot(h1, W2[...], preferred_element_type=jnp.float32)
    u2 = z2 * dinv[...]
    o_a[...] = u2[:, :F2 // 2]
    o_b[...] = u2[:, F2 // 2:]


def _tc_mm1(s1a, s1b, u1, dinv, W1, b1, W2):
    nb = NP // R1
    return pl.pallas_call(
        _tc_mm1_body,
        grid=(nb,),
        in_specs=[
            pl.BlockSpec((R1, DIN), lambda i: (i, 0)),
            pl.BlockSpec((R1, DIN), lambda i: (i, 0)),
            pl.BlockSpec((R1, DIN), lambda i: (i, 0)),
            pl.BlockSpec((R1, 1), lambda i: (i, 0)),
            pl.BlockSpec((DIN, F1), lambda i: (0, 0)),
            pl.BlockSpec((1, F1), lambda i: (0, 0)),
            pl.BlockSpec((F1, F2), lambda i: (0, 0)),
        ],
        out_specs=[
            pl.BlockSpec((R1, F2 // 2), lambda i: (i, 0)),
            pl.BlockSpec((R1, F2 // 2), lambda i: (i, 0)),
        ],
        out_shape=[
            jax.ShapeDtypeStruct((NP, F2 // 2), jnp.float32),
            jax.ShapeDtypeStruct((NP, F2 // 2), jnp.float32),
        ],
    )(s1a, s1b, u1, dinv, W1, b1, W2)


def _tc_mm3_body(s2a, s2b, u2a, u2b, dinv, b2, W3, u3_ref):
    t = jnp.concatenate([s2a[...] + u2a[...], s2b[...] + u2b[...]], axis=1)
    h2 = jnp.maximum(t * dinv[...] + b2[...], 0.0)
    z3 = jnp.dot(h2, W3[...], preferred_element_type=jnp.float32)
    u3 = z3 * dinv[...]
    u3_ref[...] = jnp.concatenate(
        [u3, jnp.zeros((u3.shape[0], DIN - F3), jnp.float32)], axis=1)


def _tc_mm3(s2a, s2b, u2a, u2b, dinv, b2, W3):
    nb = NP // R1
    return pl.pallas_call(
        _tc_mm3_body,
        grid=(nb,),
        in_specs=[
            pl.BlockSpec((R1, F2 // 2), lambda i: (i, 0)),
            pl.BlockSpec((R1, F2 // 2), lambda i: (i, 0)),
            pl.BlockSpec((R1, F2 // 2), lambda i: (i, 0)),
            pl.BlockSpec((R1, F2 // 2), lambda i: (i, 0)),
            pl.BlockSpec((R1, 1), lambda i: (i, 0)),
            pl.BlockSpec((1, F2), lambda i: (0, 0)),
            pl.BlockSpec((F2, F3), lambda i: (0, 0)),
        ],
        out_specs=pl.BlockSpec((R1, DIN), lambda i: (i, 0)),
        out_shape=jax.ShapeDtypeStruct((NP, DIN), jnp.float32),
    )(s2a, s2b, u2a, u2b, dinv, b2, W3)


def _tc_head_body(s3a, s3b, u3, dinv, b3, batch_row, Wl, bl, out_ref):
    a3 = ((s3a[...] + s3b[...] + u3[...]) * dinv[...])[:, :F3] + b3[...]
    gids = lax.broadcasted_iota(jnp.int32, (G, NP), 0)
    oneh = (batch_row[...] == gids).astype(jnp.float32)
    sums = jnp.dot(oneh, a3, preferred_element_type=jnp.float32)
    cnt = jnp.dot(oneh, jnp.ones((NP, 1), jnp.float32),
                  preferred_element_type=jnp.float32)
    pooled = sums / jnp.maximum(cnt, 1.0)
    out_ref[...] = (
        jnp.dot(pooled, Wl[...], preferred_element_type=jnp.float32) + bl[...])


def _tc_head(s3a, s3b, u3, dinv, b3, batch_row, Wl, bl):
    return pl.pallas_call(
        _tc_head_body,
        out_shape=jax.ShapeDtypeStruct((G, NC), jnp.float32),
    )(s3a, s3b, u3, dinv, b3, batch_row, Wl, bl)


def kernel(x, edge_index, batch, W1, b1, W2, b2, W3, b3, Wl, bl):
    npad = EP - E
    src2 = jnp.concatenate(
        [edge_index[0], jnp.zeros((npad,), jnp.int32)]).reshape(CH, K)
    dst2 = jnp.concatenate(
        [edge_index[1], jnp.full((npad,), NP - 1, jnp.int32)]).reshape(CH, K)
    x_pad = jnp.pad(x, ((0, NP - N), (0, 0)))
    batch_row = jnp.pad(batch, (0, NP - N), constant_values=G)[None, :]

    d0, d1 = _deg_kernel(dst2)
    dinv, u1 = _tc_scale(d0, d1, x_pad)

    s1a, s1b = _prop1(u1, u1, edge_index[0], edge_index[1])
    u2a, u2b = _tc_mm1(s1a, s1b, u1, dinv, W1, b1[None, :], W2)

    s2a, s2b = _prop2(u2a, u2b, src2, dst2)
    u3 = _tc_mm3(s2a, s2b, u2a, u2b, dinv, b2[None, :], W3)

    s3a, s3b = _prop3(u3, u3, edge_index[0], edge_index[1])
    return _tc_head(s3a, s3b, u3, dinv, b3[None, :], batch_row, Wl, bl)
